# baseline (speedup 1.0000x reference)
import jax
import jax.numpy as jnp
from jax.experimental import pallas as pl
from jax.experimental.pallas import tpu as pltpu

_EPS = 1e-5
_IMGS_PER_BLOCK = 8


def _conv_stats_kernel(b_ref, x_ref, y_ref, s_ref, q_ref, ps_ref):
    b, c, h, w = x_ref.shape
    cout = y_ref.shape[1]
    wp = ((w + 127) // 128) * 128
    zc1 = jnp.zeros((h, 1), jnp.float32)

    for img in range(b):
        for ci in range(c):
            xc = x_ref[img, ci]
            shifted = (
                jnp.concatenate([zc1, xc[:, :w - 1]], axis=1),
                xc,
                jnp.concatenate([xc[:, 1:], zc1], axis=1),
            )
            for kw in range(3):
                r0 = (ci * 3 + kw) * h
                ps_ref[r0:r0 + h, img * wp:img * wp + w] = (
                    shifted[kw].astype(jnp.bfloat16))

    s_tot = [None] * cout
    q_tot = [None] * cout
    kh_half = (c * 3 * h) // 2
    b1 = b_ref[:, :kh_half]
    b2 = b_ref[:, kh_half:]
    ychs = []
    for pair in range(b // 2):
        cols = pl.ds(pair * 2 * wp, 2 * wp)
        ychs.append(
            jnp.dot(b1, ps_ref[:kh_half, cols],
                    preferred_element_type=jnp.float32)
            + jnp.dot(b2, ps_ref[kh_half:, cols],
                      preferred_element_type=jnp.float32))
    for pair in range(b // 2):
        for half in range(2):
            img = pair * 2 + half
            for co in range(cout):
                yc = ychs[pair][co * h:(co + 1) * h, half * wp:half * wp + w]
                y_ref[img, co] = yc.astype(y_ref.dtype)
                s = jnp.sum(yc)
                q = jnp.sum(yc * yc)
                s_tot[co] = s if s_tot[co] is None else s_tot[co] + s
                q_tot[co] = q if q_tot[co] is None else q_tot[co] + q

    s_vec = jnp.stack([s_tot[co] for co in range(cout)])
    q_vec = jnp.stack([q_tot[co] for co in range(cout)])
    s_ref[0] = jnp.broadcast_to(s_vec[:, None], (cout, 128))
    q_ref[0] = jnp.broadcast_to(q_vec[:, None], (cout, 128))


def _bn_apply_kernel(sc_ref, sh_ref, y_ref, o_ref):
    b, cout = o_ref.shape[0], o_ref.shape[1]
    for img in range(b):
        for co in range(cout):
            o_ref[img, co] = (y_ref[img, co].astype(jnp.float32)
                              * sc_ref[co] + sh_ref[co])


def _make_banded_weights(weight, h):
    cout, c = weight.shape[0], weight.shape[1]
    wf = weight.astype(jnp.float32)
    rows = []
    for co in range(cout):
        blocks = []
        for ci in range(c):
            for kw in range(3):
                blk = (wf[co, ci, 0, kw] * jnp.eye(h, k=-1, dtype=jnp.float32)
                       + wf[co, ci, 1, kw] * jnp.eye(h, k=0, dtype=jnp.float32)
                       + wf[co, ci, 2, kw] * jnp.eye(h, k=1, dtype=jnp.float32))
                blocks.append(blk)
        rows.append(jnp.concatenate(blocks, axis=1))
    return jnp.concatenate(rows, axis=0).astype(jnp.bfloat16)


def kernel(x, weight, bias, gamma, beta):
    del bias
    n, c, h, w = x.shape
    cout = weight.shape[0]
    m = n * h * w
    blk = _IMGS_PER_BLOCK if n % _IMGS_PER_BLOCK == 0 else 1
    nblk = n // blk
    b_mat = _make_banded_weights(weight, h)

    y, s_p, q_p = pl.pallas_call(
        _conv_stats_kernel,
        grid=(nblk,),
        in_specs=[
            pl.BlockSpec((cout * h, c * 3 * h), lambda i: (0, 0)),
            pl.BlockSpec((blk, c, h, w), lambda i: (i, 0, 0, 0)),
        ],
        out_specs=[
            pl.BlockSpec((blk, cout, h, w), lambda i: (i, 0, 0, 0)),
            pl.BlockSpec((1, cout, 128), lambda i: (i, 0, 0)),
            pl.BlockSpec((1, cout, 128), lambda i: (i, 0, 0)),
        ],
        out_shape=(
            jax.ShapeDtypeStruct((n, cout, h, w), jnp.bfloat16),
            jax.ShapeDtypeStruct((nblk, cout, 128), jnp.float32),
            jax.ShapeDtypeStruct((nblk, cout, 128), jnp.float32),
        ),
        scratch_shapes=[
            pltpu.VMEM((c * 3 * h, blk * ((w + 127) // 128) * 128),
                       jnp.bfloat16)],
        compiler_params=pltpu.CompilerParams(
            dimension_semantics=("parallel",)),
    )(b_mat, x)

    s = jnp.sum(s_p[:, :, 0], axis=0)
    q = jnp.sum(q_p[:, :, 0], axis=0)
    mean = s / m
    var = jnp.maximum(q / m - mean * mean, 0.0)
    inv_std = jax.lax.rsqrt(var + jnp.float32(_EPS))
    scale = gamma.astype(jnp.float32) * inv_std
    shift = beta.astype(jnp.float32) - mean * scale

    out = pl.pallas_call(
        _bn_apply_kernel,
        grid=(nblk,),
        in_specs=[
            pl.BlockSpec(memory_space=pltpu.SMEM),
            pl.BlockSpec(memory_space=pltpu.SMEM),
            pl.BlockSpec((blk, cout, h, w), lambda i: (i, 0, 0, 0)),
        ],
        out_specs=pl.BlockSpec((blk, cout, h, w), lambda i: (i, 0, 0, 0)),
        out_shape=jax.ShapeDtypeStruct((n, cout, h, w), jnp.float32),
        compiler_params=pltpu.CompilerParams(
            dimension_semantics=("parallel",)),
    )(scale, shift, y)
    return out


# speedup vs baseline: 1.0672x; 1.0672x over previous
import jax
import jax.numpy as jnp
from jax.experimental import pallas as pl
from jax.experimental.pallas import tpu as pltpu

_EPS = 1e-5
_IMGS_PER_BLOCK = 8


def _conv_stats_kernel(w_ref, x_ref, y_ref, s_ref, q_ref):
    b, c, h, w = x_ref.shape
    cout = y_ref.shape[1]
    zc1 = jnp.zeros((h, 1), jnp.float32)
    zr1 = jnp.zeros((1, w), jnp.float32)

    s_tot = [None] * cout
    q_tot = [None] * cout
    for img in range(b):
        planes = []
        for ci in range(c):
            xc = x_ref[img, ci]
            planes.append([
                jnp.concatenate([zc1, xc[:, :w - 1]], axis=1),
                xc,
                jnp.concatenate([xc[:, 1:], zc1], axis=1),
            ])
        for co in range(cout):
            svs = []
            for kh in range(3):
                acc = None
                for ci in range(c):
                    for kw in range(3):
                        coeff = w_ref[((co * c + ci) * 3 + kh) * 3 + kw]
                        t = planes[ci][kw] * coeff
                        acc = t if acc is None else acc + t
                svs.append(acc)
            yc = (jnp.concatenate([zr1, svs[0][:h - 1]], axis=0) + svs[1]
                  + jnp.concatenate([svs[2][1:], zr1], axis=0))
            y_ref[img, co] = yc.astype(y_ref.dtype)
            s = jnp.sum(yc)
            q = jnp.sum(yc * yc)
            s_tot[co] = s if s_tot[co] is None else s_tot[co] + s
            q_tot[co] = q if q_tot[co] is None else q_tot[co] + q

    s_vec = jnp.stack([s_tot[co] for co in range(cout)])
    q_vec = jnp.stack([q_tot[co] for co in range(cout)])
    s_ref[0] = jnp.broadcast_to(s_vec[:, None], (cout, 128))
    q_ref[0] = jnp.broadcast_to(q_vec[:, None], (cout, 128))


def _bn_apply_kernel(m_inv_ref, g_ref, be_ref, sp_ref, qp_ref, y_ref, o_ref):
    b, cout = o_ref.shape[0], o_ref.shape[1]
    m_inv = m_inv_ref[0]
    s = jnp.sum(sp_ref[...], axis=0)
    q = jnp.sum(qp_ref[...], axis=0)
    mean = s * m_inv
    var = jnp.maximum(q * m_inv - mean * mean, 0.0)
    inv_std = jax.lax.rsqrt(var + jnp.float32(_EPS))
    scale = g_ref[...] * inv_std
    shift = be_ref[...] - mean * scale
    for img in range(b):
        for co in range(cout):
            sc = scale[co:co + 1, 0:1]
            sh = shift[co:co + 1, 0:1]
            o_ref[img, co] = (y_ref[img, co].astype(jnp.float32)
                              * sc + sh)


def kernel(x, weight, bias, gamma, beta):
    del bias
    n, c, h, w = x.shape
    cout = weight.shape[0]
    m = n * h * w
    blk = _IMGS_PER_BLOCK if n % _IMGS_PER_BLOCK == 0 else 1
    nblk = n // blk
    wf = weight.astype(jnp.float32).reshape(cout * c * 9)

    y, s_p, q_p = pl.pallas_call(
        _conv_stats_kernel,
        grid=(nblk,),
        in_specs=[
            pl.BlockSpec(memory_space=pltpu.SMEM),
            pl.BlockSpec((blk, c, h, w), lambda i: (i, 0, 0, 0)),
        ],
        out_specs=[
            pl.BlockSpec((blk, cout, h, w), lambda i: (i, 0, 0, 0)),
            pl.BlockSpec((1, cout, 128), lambda i: (i, 0, 0)),
            pl.BlockSpec((1, cout, 128), lambda i: (i, 0, 0)),
        ],
        out_shape=(
            jax.ShapeDtypeStruct((n, cout, h, w), jnp.bfloat16),
            jax.ShapeDtypeStruct((nblk, cout, 128), jnp.float32),
            jax.ShapeDtypeStruct((nblk, cout, 128), jnp.float32),
        ),
        compiler_params=pltpu.CompilerParams(
            dimension_semantics=("parallel",)),
    )(wf, x)

    m_inv = jnp.full((1,), 1.0 / m, jnp.float32)
    g_l = jnp.broadcast_to(gamma.astype(jnp.float32)[:, None], (cout, 128))
    be_l = jnp.broadcast_to(beta.astype(jnp.float32)[:, None], (cout, 128))

    out = pl.pallas_call(
        _bn_apply_kernel,
        grid=(nblk,),
        in_specs=[
            pl.BlockSpec(memory_space=pltpu.SMEM),
            pl.BlockSpec((cout, 128), lambda i: (0, 0)),
            pl.BlockSpec((cout, 128), lambda i: (0, 0)),
            pl.BlockSpec((nblk, cout, 128), lambda i: (0, 0, 0)),
            pl.BlockSpec((nblk, cout, 128), lambda i: (0, 0, 0)),
            pl.BlockSpec((blk, cout, h, w), lambda i: (i, 0, 0, 0)),
        ],
        out_specs=pl.BlockSpec((blk, cout, h, w), lambda i: (i, 0, 0, 0)),
        out_shape=jax.ShapeDtypeStruct((n, cout, h, w), jnp.float32),
        compiler_params=pltpu.CompilerParams(
            dimension_semantics=("parallel",)),
    )(m_inv, g_l, be_l, s_p, q_p, y)
    return out


# speedup vs baseline: 1.0750x; 1.0074x over previous
import jax
import jax.numpy as jnp
from jax.experimental import pallas as pl
from jax.experimental.pallas import tpu as pltpu

_EPS = 1e-5
_IMGS_PER_BLOCK = 8


def _conv_stats_kernel(w_ref, x_ref, y_ref, s_ref, q_ref):
    b, c, h, w = x_ref.shape
    cout = y_ref.shape[1]
    zc1 = jnp.zeros((h, 1), jnp.float32)
    zr1 = jnp.zeros((1, w), jnp.float32)

    s_tot = [None] * cout
    q_tot = [None] * cout
    for img in range(b):
        planes = []
        for ci in range(c):
            xc = x_ref[img, ci]
            planes.append([
                jnp.concatenate([zc1, xc[:, :w - 1]], axis=1),
                xc,
                jnp.concatenate([xc[:, 1:], zc1], axis=1),
            ])
        for co in range(cout):
            svs = []
            for kh in range(3):
                acc = None
                for ci in range(c):
                    for kw in range(3):
                        coeff = w_ref[((co * c + ci) * 3 + kh) * 3 + kw]
                        t = planes[ci][kw] * coeff
                        acc = t if acc is None else acc + t
                svs.append(acc)
            yc = (jnp.concatenate([zr1, svs[0][:h - 1]], axis=0) + svs[1]
                  + jnp.concatenate([svs[2][1:], zr1], axis=0))
            y_ref[img, co] = yc.astype(y_ref.dtype)
            s = jnp.sum(yc)
            q = jnp.sum(yc * yc)
            s_tot[co] = s if s_tot[co] is None else s_tot[co] + s
            q_tot[co] = q if q_tot[co] is None else q_tot[co] + q

    s_vec = jnp.stack([s_tot[co] for co in range(cout)])
    q_vec = jnp.stack([q_tot[co] for co in range(cout)])
    s_ref[0] = jnp.broadcast_to(s_vec[:, None], (cout, 128))
    q_ref[0] = jnp.broadcast_to(q_vec[:, None], (cout, 128))


def _bn_apply_kernel(sc_ref, sh_ref, y_ref, o_ref):
    b, cout = o_ref.shape[0], o_ref.shape[1]
    for img in range(b):
        for co in range(cout):
            o_ref[img, co] = (y_ref[img, co].astype(jnp.float32)
                              * sc_ref[co] + sh_ref[co])


def kernel(x, weight, bias, gamma, beta):
    del bias
    n, c, h, w = x.shape
    cout = weight.shape[0]
    m = n * h * w
    blk = _IMGS_PER_BLOCK if n % _IMGS_PER_BLOCK == 0 else 1
    nblk = n // blk
    wf = weight.astype(jnp.float32).reshape(cout * c * 9)

    y, s_p, q_p = pl.pallas_call(
        _conv_stats_kernel,
        grid=(nblk,),
        in_specs=[
            pl.BlockSpec(memory_space=pltpu.SMEM),
            pl.BlockSpec((blk, c, h, w), lambda i: (i, 0, 0, 0)),
        ],
        out_specs=[
            pl.BlockSpec((blk, cout, h, w), lambda i: (i, 0, 0, 0)),
            pl.BlockSpec((1, cout, 128), lambda i: (i, 0, 0)),
            pl.BlockSpec((1, cout, 128), lambda i: (i, 0, 0)),
        ],
        out_shape=(
            jax.ShapeDtypeStruct((n, cout, h, w), jnp.bfloat16),
            jax.ShapeDtypeStruct((nblk, cout, 128), jnp.float32),
            jax.ShapeDtypeStruct((nblk, cout, 128), jnp.float32),
        ),
        compiler_params=pltpu.CompilerParams(
            dimension_semantics=("parallel",)),
    )(wf, x)

    s = jnp.sum(s_p[:, :, 0], axis=0)
    q = jnp.sum(q_p[:, :, 0], axis=0)
    mean = s / m
    var = jnp.maximum(q / m - mean * mean, 0.0)
    inv_std = jax.lax.rsqrt(var + jnp.float32(_EPS))
    scale = gamma.astype(jnp.float32) * inv_std
    shift = beta.astype(jnp.float32) - mean * scale

    out = pl.pallas_call(
        _bn_apply_kernel,
        grid=(nblk,),
        in_specs=[
            pl.BlockSpec(memory_space=pltpu.SMEM),
            pl.BlockSpec(memory_space=pltpu.SMEM),
            pl.BlockSpec((blk, cout, h, w), lambda i: (i, 0, 0, 0)),
        ],
        out_specs=pl.BlockSpec((blk, cout, h, w), lambda i: (i, 0, 0, 0)),
        out_shape=jax.ShapeDtypeStruct((n, cout, h, w), jnp.float32),
        compiler_params=pltpu.CompilerParams(
            dimension_semantics=("parallel",)),
    )(scale, shift, y)
    return out
